# baseline (speedup 1.0000x reference)
import functools

import jax
import jax.numpy as jnp
import numpy as np
from jax import lax
from jax.experimental import pallas as pl
from jax.experimental.pallas import tpu as pltpu
from jax.experimental.pallas import tpu_sc as plsc

BASE_GSD_ = 10.0
D4_ = 32
MAX_SEQ_ = 24
B_, H_, W_, T_, CG_ = 16, 16, 16, 12, 4
TCG_ = T_ * CG_
NB_ = 2

SC_NC_ = 2
SC_NS_ = 16
SC_ROWS_ = 8


def _np_pos48():
    omega = 1.0 / (10000.0 ** (np.arange(D4_ // 2, dtype=np.float32)
                               / np.float32(D4_ / 2.0)))
    arg = np.arange(MAX_SEQ_, dtype=np.float32)[:, None] * omega[None, :]
    pos = np.concatenate([np.sin(arg), np.cos(arg)], axis=1)[:T_]
    out = np.zeros((T_, CG_, 128), np.float32)
    out[:, :, D4_:2 * D4_] = pos[:, None, :]
    return out.reshape(TCG_, 128)


def _np_mt128():
    angles = np.arange(0, 13, dtype=np.float32) / np.float32(12.0 / (2.0 * np.pi))
    sin_t = np.stack([np.sin(angles)] * (D4_ // 2), axis=-1)
    cos_t = np.stack([np.cos(angles)] * (D4_ // 2), axis=-1)
    mt = np.concatenate([sin_t[:-1], cos_t[:-1]], axis=-1)
    out = np.zeros((T_, 128), np.float32)
    out[:, 2 * D4_:3 * D4_] = mt
    return out


def _np_wom():
    d = D4_ // 4
    om = 1.0 / (10000.0 ** (np.arange(d, dtype=np.float32) / np.float32(d)))
    out = np.zeros((H_, W_, 128), np.float32)
    hh = np.arange(H_, dtype=np.float32)[:, None, None]
    ww = np.arange(W_, dtype=np.float32)[None, :, None]
    out[:, :, 96:104] = ww * om
    out[:, :, 104:112] = ww * om
    out[:, :, 112:120] = hh * om
    out[:, :, 120:128] = hh * om
    return out


def _np_sel(offset):
    out = np.zeros((D4_, 128), np.float32)
    out[np.arange(D4_), offset + np.arange(D4_)] = 1.0
    return out


_POS48 = _np_pos48()
_MT128 = _np_mt128()
_WOM = _np_wom()
_SEL0 = _np_sel(0)


@functools.partial(
    pl.kernel,
    mesh=plsc.VectorSubcoreMesh(core_axis_name="c", subcore_axis_name="s"),
    out_type=jax.ShapeDtypeStruct((B_ * T_, 128), jnp.float32),
    scratch_types=[
        pltpu.VMEM((SC_ROWS_,), jnp.int32),
        pltpu.VMEM((SC_ROWS_, 128), jnp.float32),
        pltpu.SemaphoreType.DMA,
    ],
)
def _month_gather_sc(months_hbm, table_hbm, out_hbm, idx_v, rows_v, sem):
    wid = lax.axis_index("s") * SC_NC_ + lax.axis_index("c")

    @pl.when(wid < (B_ * T_) // SC_ROWS_)
    def _():
        base = wid * SC_ROWS_
        pltpu.sync_copy(months_hbm.at[pl.ds(base, SC_ROWS_)], idx_v)
        pltpu.async_copy(table_hbm.at[idx_v], rows_v, sem).wait()
        pltpu.sync_copy(rows_v, out_hbm.at[pl.ds(base, SC_ROWS_)])


def _encoder_kernel(mb_ref, g_ref, ch_ref, pos48_ref, wom_ref,
                    sel0_ref, s2_ref, out_ref, spat_ref, base_ref):
    i = pl.program_id(0)

    @pl.when(i == 0)
    def _build_tables():
        arg = wom_ref[...] * g_ref[0]
        lane3 = jax.lax.broadcasted_iota(jnp.int32, (H_, W_, 128), 2)
        k = lane3 - 96
        sin_mask = jnp.logical_and(k >= 0, (k // 8) % 2 == 0)
        spat_ref[...] = jnp.where(
            lane3 >= 96,
            jnp.where(sin_mask, jnp.sin(arg), jnp.cos(arg)),
            0.0)
        chp = jnp.dot(ch_ref[...], sel0_ref[...],
                      preferred_element_type=jnp.float32)
        base_ref[...] = (jnp.broadcast_to(chp[None, :, :], (T_, CG_, 128))
                         .reshape(TCG_, 128) + pos48_ref[...])

    mb48 = jnp.broadcast_to(
        mb_ref[...].reshape(NB_, T_, 1, 128),
        (NB_, T_, CG_, 128)).reshape(NB_, TCG_, 128)
    bias = base_ref[...][None, :, :] + mb48

    spat = spat_ref[...]
    for hh in range(H_):
        out_ref[:, hh] = (s2_ref[:, hh] + bias[:, None, :, :]
                          + spat[hh][None, :, None, :])


def kernel(s2, timestamps, channel_embeds, patch_size, input_res):
    b, h, w, t, cg, e = s2.shape
    s2r = s2.reshape(b, h, w, t * cg, e)
    months_flat = timestamps[:, 1, :].reshape(b * t).astype(jnp.int32)
    g = jnp.reshape((input_res * patch_size) / BASE_GSD_, (1,)).astype(jnp.float32)

    mb32 = _month_gather_sc(months_flat, _MT128).reshape(b, t, 128)

    out = pl.pallas_call(
        _encoder_kernel,
        grid=(b // NB_,),
        in_specs=[
            pl.BlockSpec((NB_, t, 128), lambda i: (i, 0, 0)),
            pl.BlockSpec(memory_space=pltpu.SMEM),
            pl.BlockSpec((cg, D4_), lambda i: (0, 0)),
            pl.BlockSpec((TCG_, 128), lambda i: (0, 0)),
            pl.BlockSpec((h, w, 128), lambda i: (0, 0, 0)),
            pl.BlockSpec((D4_, 128), lambda i: (0, 0)),
            pl.BlockSpec((NB_, h, w, t * cg, 128), lambda i: (i, 0, 0, 0, 0)),
        ],
        out_specs=pl.BlockSpec((NB_, h, w, t * cg, 128), lambda i: (i, 0, 0, 0, 0)),
        out_shape=jax.ShapeDtypeStruct((b, h, w, t * cg, 128), jnp.float32),
        scratch_shapes=[
            pltpu.VMEM((h, w, 128), jnp.float32),
            pltpu.VMEM((TCG_, 128), jnp.float32),
        ],
        compiler_params=pltpu.CompilerParams(
            dimension_semantics=("arbitrary",),
            vmem_limit_bytes=100 * 1024 * 1024,
        ),
    )(mb32, g, channel_embeds, _POS48, _WOM, _SEL0, s2r)
    return out.reshape(b, h, w, t, cg, e)


# speedup vs baseline: 1.2798x; 1.2798x over previous
import jax
import jax.numpy as jnp
import numpy as np
from jax.experimental import pallas as pl
from jax.experimental.pallas import tpu as pltpu

BASE_GSD_ = 10.0
D4_ = 32
MAX_SEQ_ = 24
B_, H_, W_, T_, CG_ = 16, 16, 16, 12, 4
TCG_ = T_ * CG_
NB_ = 2


def _np_pos48():
    omega = 1.0 / (10000.0 ** (np.arange(D4_ // 2, dtype=np.float32)
                               / np.float32(D4_ / 2.0)))
    arg = np.arange(MAX_SEQ_, dtype=np.float32)[:, None] * omega[None, :]
    pos = np.concatenate([np.sin(arg), np.cos(arg)], axis=1)[:T_]
    out = np.zeros((T_, CG_, 128), np.float32)
    out[:, :, D4_:2 * D4_] = pos[:, None, :]
    return out.reshape(TCG_, 128)


def _np_mtab():
    angles = np.arange(0, 13, dtype=np.float32) / np.float32(12.0 / (2.0 * np.pi))
    sin_t = np.stack([np.sin(angles)] * (D4_ // 2), axis=-1)
    cos_t = np.stack([np.cos(angles)] * (D4_ // 2), axis=-1)
    mt = np.concatenate([sin_t[:-1], cos_t[:-1]], axis=-1)
    out = np.zeros((128, 128), np.float32)
    out[:T_, 2 * D4_:3 * D4_] = mt
    return out


def _np_wom():
    d = D4_ // 4
    om = 1.0 / (10000.0 ** (np.arange(d, dtype=np.float32) / np.float32(d)))
    out = np.zeros((H_, W_, 128), np.float32)
    hh = np.arange(H_, dtype=np.float32)[:, None, None]
    ww = np.arange(W_, dtype=np.float32)[None, :, None]
    out[:, :, 96:104] = ww * om
    out[:, :, 104:112] = ww * om
    out[:, :, 112:120] = hh * om
    out[:, :, 120:128] = hh * om
    return out


def _np_chsel():
    out = np.zeros((D4_, 128), np.float32)
    out[np.arange(D4_), np.arange(D4_)] = 1.0
    return out


_POS48 = _np_pos48()
_MTAB = _np_mtab()
_WOM = _np_wom()
_CHSEL = _np_chsel()


def _encoder_kernel(ts_ref, g_ref, ch_ref, pos48_ref, mtab_ref, wom_ref,
                    chsel_ref, s2_ref, out_ref, spat_ref, base_ref):
    i = pl.program_id(0)

    arg = wom_ref[...] * g_ref[0]
    lane3 = jax.lax.broadcasted_iota(jnp.int32, (H_, W_, 128), 2)
    k = lane3 - 96
    sin_mask = jnp.logical_and(k >= 0, (k // 8) % 2 == 0)
    spat_ref[...] = jnp.where(
        lane3 >= 96,
        jnp.where(sin_mask, jnp.sin(arg), jnp.cos(arg)),
        0.0)
    chp = jnp.dot(ch_ref[...], chsel_ref[...],
                  preferred_element_type=jnp.float32)
    base_ref[...] = (jnp.broadcast_to(chp[None, :, :], (T_, CG_, 128))
                     .reshape(TCG_, 128) + pos48_ref[...])

    lane = jax.lax.broadcasted_iota(jnp.int32, (1, 128), 1)
    rows = []
    for bb in range(NB_):
        for t in range(T_):
            m = ts_ref[i * NB_ + bb, 1, t]
            rows.append((lane == m).astype(jnp.float32))
    onehot = jnp.concatenate(rows, axis=0)
    mb = jnp.dot(onehot, mtab_ref[...],
                 preferred_element_type=jnp.float32)
    mb48 = jnp.broadcast_to(
        mb.reshape(NB_, T_, 1, 128), (NB_, T_, CG_, 128)).reshape(NB_, TCG_, 128)
    bias = base_ref[...][None, :, :] + mb48

    spat = spat_ref[...]
    for hh in range(H_):
        out_ref[:, hh] = (s2_ref[:, hh] + bias[:, None, :, :]
                          + spat[hh][None, :, None, :])


def kernel(s2, timestamps, channel_embeds, patch_size, input_res):
    b, h, w, t, cg, e = s2.shape
    s2r = s2.reshape(b, h, w, t * cg, e)
    ts = timestamps.astype(jnp.int32)
    g = jnp.reshape((input_res * patch_size) / BASE_GSD_, (1,)).astype(jnp.float32)

    out = pl.pallas_call(
        _encoder_kernel,
        grid=(b // NB_,),
        in_specs=[
            pl.BlockSpec(memory_space=pltpu.SMEM),
            pl.BlockSpec(memory_space=pltpu.SMEM),
            pl.BlockSpec((cg, D4_), lambda i: (0, 0)),
            pl.BlockSpec((TCG_, 128), lambda i: (0, 0)),
            pl.BlockSpec((128, 128), lambda i: (0, 0)),
            pl.BlockSpec((h, w, 128), lambda i: (0, 0, 0)),
            pl.BlockSpec((D4_, 128), lambda i: (0, 0)),
            pl.BlockSpec((NB_, h, w, t * cg, 128), lambda i: (i, 0, 0, 0, 0)),
        ],
        out_specs=pl.BlockSpec((NB_, h, w, t * cg, 128), lambda i: (i, 0, 0, 0, 0)),
        out_shape=jax.ShapeDtypeStruct((b, h, w, t * cg, 128), jnp.float32),
        scratch_shapes=[
            pltpu.VMEM((h, w, 128), jnp.float32),
            pltpu.VMEM((TCG_, 128), jnp.float32),
        ],
        compiler_params=pltpu.CompilerParams(
            dimension_semantics=("parallel",),
            vmem_limit_bytes=100 * 1024 * 1024,
        ),
    )(ts, g, channel_embeds, _POS48, _MTAB, _WOM, _CHSEL, s2r)
    return out.reshape(b, h, w, t, cg, e)


# speedup vs baseline: 1.2899x; 1.0078x over previous
import jax
import jax.numpy as jnp
import numpy as np
from jax.experimental import pallas as pl
from jax.experimental.pallas import tpu as pltpu

BASE_GSD_ = 10.0
D4_ = 32
MAX_SEQ_ = 24
B_, H_, W_, T_, CG_ = 16, 16, 16, 12, 4
TCG_ = T_ * CG_
NB_ = 2


def _np_pos48():
    omega = 1.0 / (10000.0 ** (np.arange(D4_ // 2, dtype=np.float32)
                               / np.float32(D4_ / 2.0)))
    arg = np.arange(MAX_SEQ_, dtype=np.float32)[:, None] * omega[None, :]
    pos = np.concatenate([np.sin(arg), np.cos(arg)], axis=1)[:T_]
    out = np.zeros((T_, CG_, 128), np.float32)
    out[:, :, D4_:2 * D4_] = pos[:, None, :]
    return out.reshape(TCG_, 128)


def _np_mtab():
    angles = np.arange(0, 13, dtype=np.float32) / np.float32(12.0 / (2.0 * np.pi))
    sin_t = np.stack([np.sin(angles)] * (D4_ // 2), axis=-1)
    cos_t = np.stack([np.cos(angles)] * (D4_ // 2), axis=-1)
    mt = np.concatenate([sin_t[:-1], cos_t[:-1]], axis=-1)
    out = np.zeros((128, 128), np.float32)
    out[:T_, 2 * D4_:3 * D4_] = mt
    return out


def _np_wom():
    d = D4_ // 4
    om = 1.0 / (10000.0 ** (np.arange(d, dtype=np.float32) / np.float32(d)))
    out = np.zeros((H_, W_, 128), np.float32)
    hh = np.arange(H_, dtype=np.float32)[:, None, None]
    ww = np.arange(W_, dtype=np.float32)[None, :, None]
    out[:, :, 96:104] = ww * om
    out[:, :, 104:112] = ww * om
    out[:, :, 112:120] = hh * om
    out[:, :, 120:128] = hh * om
    return out


def _np_chsel():
    out = np.zeros((D4_, 128), np.float32)
    out[np.arange(D4_), np.arange(D4_)] = 1.0
    return out


_POS48 = _np_pos48()
_MTAB = _np_mtab()
_WOM = _np_wom()
_CHSEL = _np_chsel()


def _encoder_kernel(ts_ref, ir_ref, ps_ref, ch_ref, pos48_ref, mtab_ref,
                    wom_ref, chsel_ref, s2_ref, out_ref, spat_ref, base_ref):
    i = pl.program_id(0)

    @pl.when(i == 0)
    def _build_tables():
        g = (ir_ref[0] * ps_ref[0]).astype(jnp.float32) / BASE_GSD_
        arg = wom_ref[...] * g
        lane3 = jax.lax.broadcasted_iota(jnp.int32, (H_, W_, 128), 2)
        k = lane3 - 96
        sin_mask = jnp.logical_and(k >= 0, (k // 8) % 2 == 0)
        spat_ref[...] = jnp.where(
            lane3 >= 96,
            jnp.where(sin_mask, jnp.sin(arg), jnp.cos(arg)),
            0.0)
        chp = jnp.dot(ch_ref[...], chsel_ref[...],
                      preferred_element_type=jnp.float32)
        base_ref[...] = (jnp.broadcast_to(chp[None, :, :], (T_, CG_, 128))
                         .reshape(TCG_, 128) + pos48_ref[...])

    lane = jax.lax.broadcasted_iota(jnp.int32, (1, 128), 1)
    rows = []
    for bb in range(NB_):
        for t in range(T_):
            m = ts_ref[i * NB_ + bb, 1, t]
            rows.append((lane == m).astype(jnp.float32))
    onehot = jnp.concatenate(rows, axis=0)
    mb = jnp.dot(onehot, mtab_ref[...],
                 preferred_element_type=jnp.float32)
    mb48 = jnp.broadcast_to(
        mb.reshape(NB_, T_, 1, 128), (NB_, T_, CG_, 128)).reshape(NB_, TCG_, 128)
    bias = base_ref[...][None, :, :] + mb48

    spat = spat_ref[...]
    for hh in range(H_):
        out_ref[:, hh] = (s2_ref[:, hh] + bias[:, None, :, :]
                          + spat[hh][None, :, None, :])


def kernel(s2, timestamps, channel_embeds, patch_size, input_res):
    b, h, w, t, cg, e = s2.shape
    s2r = s2.reshape(b, h, w, t * cg, e)
    ts = timestamps.astype(jnp.int32)
    ir = jnp.reshape(input_res, (1,)).astype(jnp.int32)
    ps = jnp.reshape(patch_size, (1,)).astype(jnp.int32)

    out = pl.pallas_call(
        _encoder_kernel,
        grid=(b // NB_,),
        in_specs=[
            pl.BlockSpec(memory_space=pltpu.SMEM),
            pl.BlockSpec(memory_space=pltpu.SMEM),
            pl.BlockSpec(memory_space=pltpu.SMEM),
            pl.BlockSpec((cg, D4_), lambda i: (0, 0)),
            pl.BlockSpec((TCG_, 128), lambda i: (0, 0)),
            pl.BlockSpec((128, 128), lambda i: (0, 0)),
            pl.BlockSpec((h, w, 128), lambda i: (0, 0, 0)),
            pl.BlockSpec((D4_, 128), lambda i: (0, 0)),
            pl.BlockSpec((NB_, h, w, t * cg, 128), lambda i: (i, 0, 0, 0, 0)),
        ],
        out_specs=pl.BlockSpec((NB_, h, w, t * cg, 128), lambda i: (i, 0, 0, 0, 0)),
        out_shape=jax.ShapeDtypeStruct((b, h, w, t * cg, 128), jnp.float32),
        scratch_shapes=[
            pltpu.VMEM((h, w, 128), jnp.float32),
            pltpu.VMEM((TCG_, 128), jnp.float32),
        ],
        compiler_params=pltpu.CompilerParams(
            dimension_semantics=("arbitrary",),
            vmem_limit_bytes=100 * 1024 * 1024,
        ),
    )(ts, ir, ps, channel_embeds, _POS48, _MTAB, _WOM, _CHSEL, s2r)
    return out.reshape(b, h, w, t, cg, e)
